# baseline (speedup 1.0000x reference)
import functools
import math

import jax
import jax.numpy as jnp
from jax.experimental import pallas as pl
from jax.experimental.pallas import tpu as pltpu

S = 2048
D = 1024
FF = 4096
E = 6
V = 4
K = 2
FIXED = E - V
LANES = 128
FF_CHUNK = 1024
NFF = FF // FF_CHUNK
BS = 512
SB = S // BS
LNB = 256
_INV_SQRT2 = 0.7071067811865476


def _router_kernel(x_ref, wr_ref, combw_ref, meta_ref, posm_ref, cntv_ref,
                   aux_ref, l_ref):
    xs = x_ref[...]
    logits = jax.lax.dot_general(
        xs, wr_ref[...].astype(jnp.bfloat16),
        (((1,), (0,)), ((), ())),
        preferred_element_type=jnp.float32)
    lane = jax.lax.broadcasted_iota(jnp.int32, (S, LANES), 1)
    valid = lane < V
    neg = jnp.float32(-1e30)
    logits = jnp.where(valid, logits, neg)
    m = jnp.max(logits, axis=1, keepdims=True)
    ex = jnp.where(valid, jnp.exp(logits - m), 0.0)
    denom = jnp.sum(ex, axis=1, keepdims=True)
    probs = ex / denom
    p1 = jnp.max(probs, axis=1, keepdims=True)
    big = jnp.int32(LANES)
    i1 = jnp.min(jnp.where((probs == p1) & valid, lane, big), axis=1,
                 keepdims=True)
    rest = jnp.where(lane == i1, neg, probs)
    p2 = jnp.max(rest, axis=1, keepdims=True)
    i2 = jnp.min(jnp.where((rest == p2) & valid, lane, big), axis=1,
                 keepdims=True)
    wsum = p1 + p2
    w1n = p1 / wsum
    w2n = p2 / wsum
    sel1 = lane == i1
    sel2 = lane == i2
    sel = sel1 | sel2
    sub = jax.lax.broadcasted_iota(jnp.int32, (S, S), 0)
    ln2 = jax.lax.broadcasted_iota(jnp.int32, (S, S), 1)
    l_ref[...] = jnp.where(ln2 <= sub, jnp.float32(1.0), jnp.float32(0.0))
    maskb = jnp.where(sel, jnp.float32(1.0), jnp.float32(0.0))
    posf = jax.lax.dot_general(l_ref[...], maskb, (((1,), (0,)), ((), ())),
                               preferred_element_type=jnp.float32)
    pos0 = posf.astype(jnp.int32) - 1
    eA = jnp.minimum(i1, i2)
    eB = jnp.maximum(i1, i2)
    posA = jnp.sum(jnp.where(lane == eA, pos0, 0), axis=1, keepdims=True)
    posB = jnp.sum(jnp.where(lane == eB, pos0, 0), axis=1, keepdims=True)
    fa = eA * S + posA
    fb = eB * S + posB
    wA = jnp.where(eA == i1, w1n, w2n)
    wB = jnp.where(eB == i1, w1n, w2n)
    is0 = (lane == 0).astype(jnp.int32)
    is1 = (lane == 1).astype(jnp.int32)
    meta_ref[...] = fa * is0 + fb * is1
    combw_ref[...] = wA * is0.astype(jnp.float32) + wB * is1.astype(
        jnp.float32)
    posm_ref[...] = jnp.where(sel, pos0, jnp.int32(S))
    counts = jnp.sum(sel1.astype(jnp.float32) + sel2.astype(jnp.float32),
                     axis=0, keepdims=True)
    cntv_ref[...] = jnp.broadcast_to(counts, (8, LANES))
    psum = jnp.sum(probs, axis=0, keepdims=True)
    density = psum / jnp.float32(S)
    usage = counts / jnp.float32(S)
    balance = jnp.sum(density * usage) * jnp.float32(E)
    important = jnp.sum(psum * psum) / jnp.float32(E)
    aux_ref[0, 0] = balance + important


def _moe_kernel(nblk_ref, xf_ref, xg_ref, w1_ref, b1_ref, w2_ref, b2_ref,
                fx_ref, rv_ref, acc_ref, w1b_ref, w2b_ref):
    g = pl.program_id(0)
    f = pl.program_id(1)
    sb = pl.program_id(2)
    rows = pl.ds(sb * BS, BS)

    def compute(xb):
        @pl.when(sb == 0)
        def _():
            w1b_ref[...] = w1_ref[0].astype(jnp.bfloat16)
            w2b_ref[...] = w2_ref[0].astype(jnp.bfloat16)

        h = jax.lax.dot_general(xb, w1b_ref[...], (((1,), (0,)), ((), ())),
                                preferred_element_type=jnp.float32
                                ).astype(jnp.bfloat16)
        h = h + b1_ref[pl.ds(g, 1), pl.ds(f * FF_CHUNK, FF_CHUNK)].astype(
            jnp.bfloat16)
        h = (jnp.bfloat16(0.5) * h
             * (jnp.bfloat16(1.0)
                + jax.lax.erf(h * jnp.bfloat16(_INV_SQRT2))))
        contrib = jax.lax.dot_general(h, w2b_ref[...], (((1,), (0,)), ((), ())),
                                      preferred_element_type=jnp.float32)

        @pl.when((f == 0) & (g != 1))
        def _():
            acc_ref[rows, :] = contrib + b2_ref[pl.ds(g, 1), :]

        @pl.when((f == 0) & (g == 1))
        def _():
            acc_ref[rows, :] += contrib + b2_ref[pl.ds(g, 1), :]

        @pl.when(f != 0)
        def _():
            acc_ref[rows, :] += contrib

        @pl.when((f == NFF - 1) & (g == 1))
        def _():
            fx_ref[...] = acc_ref[rows, :]

        @pl.when((f == NFF - 1) & (g >= FIXED))
        def _():
            rv_ref[...] = acc_ref[rows, :].astype(jnp.bfloat16)

    @pl.when(sb < nblk_ref[g])
    def _():
        @pl.when(g < FIXED)
        def _():
            compute(xf_ref[...])

        @pl.when(g >= FIXED)
        def _():
            compute(xg_ref[...])


def _ln_kernel(fx_ref, gv_ref, cw_ref, gm_ref, bt_ref, y_ref):
    wa = cw_ref[:, 0:1]
    wb = cw_ref[:, 1:2]
    a = (fx_ref[...] + wa * gv_ref[:, :D].astype(jnp.float32)
         + wb * gv_ref[:, D:].astype(jnp.float32))
    mu = jnp.mean(a, axis=1, keepdims=True)
    var = jnp.mean((a - mu) ** 2, axis=1, keepdims=True)
    y_ref[...] = (a - mu) * jax.lax.rsqrt(var + 1e-5) * gm_ref[...] + bt_ref[...]


@jax.jit
def kernel(x, Wr, W1, b1, W2, b2, gamma, beta):
    xs = x.reshape(S, D).astype(jnp.bfloat16)
    wr_pad = jnp.zeros((D, LANES), jnp.float32).at[:, :V].set(Wr)

    combw, meta, posm, cntv, aux = pl.pallas_call(
        _router_kernel,
        out_shape=[
            jax.ShapeDtypeStruct((S, LANES), jnp.float32),
            jax.ShapeDtypeStruct((S, LANES), jnp.int32),
            jax.ShapeDtypeStruct((S, LANES), jnp.int32),
            jax.ShapeDtypeStruct((8, LANES), jnp.float32),
            jax.ShapeDtypeStruct((1, 1), jnp.float32),
        ],
        in_specs=[
            pl.BlockSpec((S, D), lambda: (0, 0)),
            pl.BlockSpec((D, LANES), lambda: (0, 0)),
        ],
        out_specs=[
            pl.BlockSpec((S, LANES), lambda: (0, 0)),
            pl.BlockSpec((S, LANES), lambda: (0, 0)),
            pl.BlockSpec((S, LANES), lambda: (0, 0)),
            pl.BlockSpec((8, LANES), lambda: (0, 0)),
            pl.BlockSpec(memory_space=pltpu.SMEM),
        ],
        scratch_shapes=[pltpu.VMEM((S, S), jnp.float32)],
    )(xs, wr_pad)

    cnt = cntv[0, :V].astype(jnp.int32)
    tok = jnp.arange(S, dtype=jnp.int32)
    eidx = jnp.broadcast_to(jnp.arange(V, dtype=jnp.int32)[None, :], (S, V))
    safe = posm[:, :V]
    ids = jnp.zeros((V, S + 1), jnp.int32).at[
        eidx.ravel(), safe.ravel()].set(
        jnp.broadcast_to(tok[:, None], (S, V)).ravel(), mode="drop")[:, :S]
    xg = jnp.take(xs, ids.reshape(V * S), axis=0)
    nblk = jnp.concatenate([
        jnp.full((FIXED,), SB, jnp.int32),
        (cnt + BS - 1) // BS,
    ]).astype(jnp.int32)

    fx, rv = pl.pallas_call(
        _moe_kernel,
        grid=(E, NFF, SB),
        out_shape=[
            jax.ShapeDtypeStruct(((SB + 1) * BS, D), jnp.float32),
            jax.ShapeDtypeStruct(((V * SB + 1) * BS, D), jnp.bfloat16),
        ],
        in_specs=[
            pl.BlockSpec(memory_space=pltpu.SMEM),
            pl.BlockSpec((BS, D),
                         lambda g, f, sb: (jnp.where(g < FIXED, sb, 0), 0)),
            pl.BlockSpec((BS, D),
                         lambda g, f, sb: (
                             jnp.where(g >= FIXED,
                                       (g - FIXED) * SB + sb, 0), 0)),
            pl.BlockSpec((1, D, FF_CHUNK), lambda g, f, sb: (g, 0, f)),
            pl.BlockSpec((E, FF), lambda g, f, sb: (0, 0)),
            pl.BlockSpec((1, FF_CHUNK, D), lambda g, f, sb: (g, f, 0)),
            pl.BlockSpec((E, D), lambda g, f, sb: (0, 0)),
        ],
        out_specs=[
            pl.BlockSpec((BS, D),
                         lambda g, f, sb: (
                             jnp.where((g == 1) & (f == NFF - 1), sb, SB),
                             0)),
            pl.BlockSpec((BS, D),
                         lambda g, f, sb: (
                             jnp.where((g >= FIXED) & (f == NFF - 1),
                                       (g - FIXED) * SB + sb, V * SB), 0)),
        ],
        scratch_shapes=[
            pltpu.VMEM((S, D), jnp.float32),
            pltpu.VMEM((D, FF_CHUNK), jnp.bfloat16),
            pltpu.VMEM((FF_CHUNK, D), jnp.bfloat16),
        ],
    )(nblk, xs, xg, W1, b1, W2, b2)

    gv = jnp.take(rv, meta[:, :K].reshape(S * K), axis=0).reshape(S, K * D)

    y = pl.pallas_call(
        _ln_kernel,
        grid=(S // LNB,),
        out_shape=jax.ShapeDtypeStruct((S, D), jnp.float32),
        in_specs=[
            pl.BlockSpec((LNB, D), lambda i: (i, 0)),
            pl.BlockSpec((LNB, K * D), lambda i: (i, 0)),
            pl.BlockSpec((LNB, LANES), lambda i: (i, 0)),
            pl.BlockSpec((1, D), lambda i: (0, 0)),
            pl.BlockSpec((1, D), lambda i: (0, 0)),
        ],
        out_specs=pl.BlockSpec((LNB, D), lambda i: (i, 0)),
    )(fx[:S], gv, combw, gamma.reshape(1, D), beta.reshape(1, D))

    return y.reshape(1, S, D), aux[0, 0]


# speedup vs baseline: 1.9157x; 1.9157x over previous
import functools
import math

import jax
import jax.numpy as jnp
from jax.experimental import pallas as pl
from jax.experimental.pallas import tpu as pltpu

S = 2048
D = 1024
FF = 4096
E = 6
V = 4
K = 2
FIXED = E - V
LANES = 128
FF_CHUNK = 1024
NFF = FF // FF_CHUNK
_INV_SQRT2 = 0.7071067811865476


def _router_kernel(x_ref, wr_ref, w_ref, aux_ref):
    xs = x_ref[...]
    logits = jax.lax.dot_general(
        xs, wr_ref[...].astype(jnp.bfloat16),
        (((1,), (0,)), ((), ())),
        preferred_element_type=jnp.float32)
    lane = jax.lax.broadcasted_iota(jnp.int32, (S, LANES), 1)
    valid = lane < V
    neg = jnp.float32(-1e30)
    logits = jnp.where(valid, logits, neg)
    m = jnp.max(logits, axis=1, keepdims=True)
    ex = jnp.where(valid, jnp.exp(logits - m), 0.0)
    denom = jnp.sum(ex, axis=1, keepdims=True)
    probs = ex / denom
    p1 = jnp.max(probs, axis=1, keepdims=True)
    big = jnp.int32(LANES)
    i1 = jnp.min(jnp.where((probs == p1) & valid, lane, big), axis=1,
                 keepdims=True)
    rest = jnp.where(lane == i1, neg, probs)
    p2 = jnp.max(rest, axis=1, keepdims=True)
    i2 = jnp.min(jnp.where((rest == p2) & valid, lane, big), axis=1,
                 keepdims=True)
    wsum = p1 + p2
    w1 = p1 / wsum
    w2 = p2 / wsum
    sel1 = lane == i1
    sel2 = lane == i2
    w_ref[...] = jnp.where(sel1, w1, 0.0) + jnp.where(sel2, w2, 0.0)
    counts = jnp.sum(sel1.astype(jnp.float32) + sel2.astype(jnp.float32),
                     axis=0, keepdims=True)
    psum = jnp.sum(probs, axis=0, keepdims=True)
    density = psum / jnp.float32(S)
    usage = counts / jnp.float32(S)
    balance = jnp.sum(density * usage) * jnp.float32(E)
    important = jnp.sum(psum * psum) / jnp.float32(E)
    aux_ref[0, 0] = balance + important


def _moe_kernel(x_ref, w1_ref, b1_ref, w2_ref, b2_ref, w_ref, g_ref, bt_ref,
                y_ref):
    e = pl.program_id(0)
    f = pl.program_id(1)

    @pl.when((e == 0) & (f == 0))
    def _():
        y_ref[...] = jnp.zeros_like(y_ref)

    xb = x_ref[...]
    w1c = w1_ref[0].astype(jnp.bfloat16)
    h = jax.lax.dot_general(xb, w1c, (((1,), (0,)), ((), ())),
                            preferred_element_type=jnp.float32
                            ).astype(jnp.bfloat16)
    h = h + b1_ref[pl.ds(e, 1), pl.ds(f * FF_CHUNK, FF_CHUNK)].astype(
        jnp.bfloat16)
    h = (jnp.bfloat16(0.5) * h
         * (jnp.bfloat16(1.0)
            + jax.lax.erf(h * jnp.bfloat16(_INV_SQRT2))))

    lane = jax.lax.broadcasted_iota(jnp.int32, (S, LANES), 1)
    wsel = jnp.sum(jnp.where(lane == e - FIXED, w_ref[...], 0.0), axis=1,
                   keepdims=True)
    wcol = jnp.where(e < FIXED, 1.0, wsel)
    hw = h * wcol.astype(jnp.bfloat16)
    w2c = w2_ref[0].astype(jnp.bfloat16)
    y_ref[...] += jax.lax.dot_general(hw, w2c, (((1,), (0,)), ((), ())),
                                      preferred_element_type=jnp.float32)

    @pl.when(f == 0)
    def _():
        y_ref[...] += wcol * b2_ref[pl.ds(e, 1), :]

    @pl.when((e == E - 1) & (f == NFF - 1))
    def _():
        acc = y_ref[...]
        mu = jnp.mean(acc, axis=1, keepdims=True)
        var = jnp.mean((acc - mu) ** 2, axis=1, keepdims=True)
        y_ref[...] = ((acc - mu) * jax.lax.rsqrt(var + 1e-5) * g_ref[...]
                      + bt_ref[...])


@jax.jit
def kernel(x, Wr, W1, b1, W2, b2, gamma, beta):
    xs = x.reshape(S, D).astype(jnp.bfloat16)
    wr_pad = jnp.zeros((D, LANES), jnp.float32).at[:, :V].set(Wr)

    w_var, aux = pl.pallas_call(
        _router_kernel,
        out_shape=[
            jax.ShapeDtypeStruct((S, LANES), jnp.float32),
            jax.ShapeDtypeStruct((1, 1), jnp.float32),
        ],
        in_specs=[
            pl.BlockSpec((S, D), lambda: (0, 0)),
            pl.BlockSpec((D, LANES), lambda: (0, 0)),
        ],
        out_specs=[
            pl.BlockSpec((S, LANES), lambda: (0, 0)),
            pl.BlockSpec(memory_space=pltpu.SMEM),
        ],
    )(xs, wr_pad)

    y = pl.pallas_call(
        _moe_kernel,
        grid=(E, NFF),
        out_shape=jax.ShapeDtypeStruct((S, D), jnp.float32),
        in_specs=[
            pl.BlockSpec((S, D), lambda e, f: (0, 0)),
            pl.BlockSpec((1, D, FF_CHUNK), lambda e, f: (e, 0, f)),
            pl.BlockSpec((E, FF), lambda e, f: (0, 0)),
            pl.BlockSpec((1, FF_CHUNK, D), lambda e, f: (e, f, 0)),
            pl.BlockSpec((E, D), lambda e, f: (0, 0)),
            pl.BlockSpec((S, LANES), lambda e, f: (0, 0)),
            pl.BlockSpec((1, D), lambda e, f: (0, 0)),
            pl.BlockSpec((1, D), lambda e, f: (0, 0)),
        ],
        out_specs=pl.BlockSpec((S, D), lambda e, f: (0, 0)),
    )(xs, W1, b1, W2, b2, w_var, gamma.reshape(1, D),
      beta.reshape(1, D))

    return y.reshape(1, S, D), aux[0, 0]


# speedup vs baseline: 1.9429x; 1.0142x over previous
import functools
import math

import jax
import jax.numpy as jnp
from jax.experimental import pallas as pl
from jax.experimental.pallas import tpu as pltpu

S = 2048
D = 1024
FF = 4096
E = 6
V = 4
K = 2
FIXED = E - V
LANES = 128
FF_CHUNK = 1024
NFF = FF // FF_CHUNK
_INV_SQRT2 = 0.7071067811865476


def _moe_kernel(x_ref, wr_ref, w1_ref, b1_ref, w2_ref, b2_ref, g_ref, bt_ref,
                y_ref, aux_ref, w_sc):
    e = pl.program_id(0)
    f = pl.program_id(1)

    @pl.when((e == 0) & (f == 0))
    def _():
        logits = jax.lax.dot_general(
            x_ref[...], wr_ref[...].astype(jnp.bfloat16),
            (((1,), (0,)), ((), ())),
            preferred_element_type=jnp.float32)
        lane = jax.lax.broadcasted_iota(jnp.int32, (S, LANES), 1)
        valid = lane < V
        neg = jnp.float32(-1e30)
        logits = jnp.where(valid, logits, neg)
        m = jnp.max(logits, axis=1, keepdims=True)
        ex = jnp.where(valid, jnp.exp(logits - m), 0.0)
        denom = jnp.sum(ex, axis=1, keepdims=True)
        probs = ex / denom
        p1 = jnp.max(probs, axis=1, keepdims=True)
        big = jnp.int32(LANES)
        i1 = jnp.min(jnp.where((probs == p1) & valid, lane, big), axis=1,
                     keepdims=True)
        rest = jnp.where(lane == i1, neg, probs)
        p2 = jnp.max(rest, axis=1, keepdims=True)
        i2 = jnp.min(jnp.where((rest == p2) & valid, lane, big), axis=1,
                     keepdims=True)
        wsum = p1 + p2
        sel1 = lane == i1
        sel2 = lane == i2
        w_sc[...] = (jnp.where(sel1, p1 / wsum, 0.0)
                     + jnp.where(sel2, p2 / wsum, 0.0))
        counts = jnp.sum(sel1.astype(jnp.float32) + sel2.astype(jnp.float32),
                         axis=0, keepdims=True)
        psum = jnp.sum(probs, axis=0, keepdims=True)
        density = psum / jnp.float32(S)
        usage = counts / jnp.float32(S)
        balance = jnp.sum(density * usage) * jnp.float32(E)
        important = jnp.sum(psum * psum) / jnp.float32(E)
        aux_ref[0, 0] = balance + important
        y_ref[...] = jnp.zeros_like(y_ref)

    xb = x_ref[...]
    w1c = w1_ref[0].astype(jnp.bfloat16)
    h = jax.lax.dot_general(xb, w1c, (((1,), (0,)), ((), ())),
                            preferred_element_type=jnp.float32
                            ).astype(jnp.bfloat16)
    h = h + b1_ref[pl.ds(e, 1), pl.ds(f * FF_CHUNK, FF_CHUNK)].astype(
        jnp.bfloat16)
    h = (jnp.bfloat16(0.5) * h
         * (jnp.bfloat16(1.0)
            + jax.lax.erf(h * jnp.bfloat16(_INV_SQRT2))))

    lane = jax.lax.broadcasted_iota(jnp.int32, (S, LANES), 1)
    wsel = jnp.sum(jnp.where(lane == e - FIXED, w_sc[...], 0.0), axis=1,
                   keepdims=True)
    wcol = jnp.where(e < FIXED, 1.0, wsel)
    hw = h * wcol.astype(jnp.bfloat16)
    w2c = w2_ref[0].astype(jnp.bfloat16)
    y_ref[...] += jax.lax.dot_general(hw, w2c, (((1,), (0,)), ((), ())),
                                      preferred_element_type=jnp.float32)

    @pl.when(f == 0)
    def _():
        y_ref[...] += wcol * b2_ref[pl.ds(e, 1), :]

    @pl.when((e == E - 1) & (f == NFF - 1))
    def _():
        acc = y_ref[...]
        mu = jnp.mean(acc, axis=1, keepdims=True)
        var = jnp.mean((acc - mu) ** 2, axis=1, keepdims=True)
        y_ref[...] = ((acc - mu) * jax.lax.rsqrt(var + 1e-5) * g_ref[...]
                      + bt_ref[...])


@jax.jit
def kernel(x, Wr, W1, b1, W2, b2, gamma, beta):
    xs = x.reshape(S, D).astype(jnp.bfloat16)
    wr_pad = jnp.zeros((D, LANES), jnp.float32).at[:, :V].set(Wr)

    y, aux = pl.pallas_call(
        _moe_kernel,
        grid=(E, NFF),
        out_shape=[
            jax.ShapeDtypeStruct((S, D), jnp.float32),
            jax.ShapeDtypeStruct((1, 1), jnp.float32),
        ],
        in_specs=[
            pl.BlockSpec((S, D), lambda e, f: (0, 0)),
            pl.BlockSpec((D, LANES), lambda e, f: (0, 0)),
            pl.BlockSpec((1, D, FF_CHUNK), lambda e, f: (e, 0, f)),
            pl.BlockSpec((E, FF), lambda e, f: (0, 0)),
            pl.BlockSpec((1, FF_CHUNK, D), lambda e, f: (e, f, 0)),
            pl.BlockSpec((E, D), lambda e, f: (0, 0)),
            pl.BlockSpec((1, D), lambda e, f: (0, 0)),
            pl.BlockSpec((1, D), lambda e, f: (0, 0)),
        ],
        out_specs=[
            pl.BlockSpec((S, D), lambda e, f: (0, 0)),
            pl.BlockSpec(memory_space=pltpu.SMEM),
        ],
        scratch_shapes=[pltpu.VMEM((S, LANES), jnp.float32)],
    )(xs, wr_pad, W1, b1, W2, b2, gamma.reshape(1, D), beta.reshape(1, D))

    return y.reshape(1, S, D), aux[0, 0]
